# baseline (speedup 1.0000x reference)
import numpy as np
import jax
import jax.numpy as jnp
from jax import lax
from jax.experimental import pallas as pl
from jax.experimental.pallas import tpu as pltpu
from jax.experimental.pallas import tpu_sc as plsc

N_NODES = 10000
D = 128
DH = D // 2
N_EDGES = 320000

NPAD = 10240
NC, NS = 2, 16
NW = NC * NS
E_PER_TILE = N_EDGES // NW
K = 128
NCHUNK = 160
EPT_PAD = NCHUNK * K
NBUF = 2
RPT = NPAD // NS

_p = []
for _h in range(DH // 32):
    _p += [_h * 32 + 2 * _r for _r in range(16)]
    _p += [_h * 32 + 2 * _r + 1 for _r in range(16)]
PERM_H = np.array(_p, dtype=np.int32)
PERM_F = np.concatenate([PERM_H, DH + PERM_H])
INV_PERM_F = np.argsort(PERM_F)

_mesh = plsc.VectorSubcoreMesh(core_axis_name="c", subcore_axis_name="s")


def _deg_body(dst_hbm, out_hbm, didx_v, deg_v, row_v, res_v, stage_sh):
    cid = lax.axis_index("c")
    sid = lax.axis_index("s")
    g = cid * NS + sid

    zeros16 = jnp.zeros((16,), jnp.float32)

    def zero_deg(i, carry):
        deg_v[pl.ds(i * 16, 16)] = zeros16
        return carry

    lax.fori_loop(0, NPAD // 16, zero_deg, 0)

    pltpu.sync_copy(dst_hbm.at[pl.ds(g * E_PER_TILE, E_PER_TILE)], didx_v)

    ones16 = jnp.ones((16,), jnp.float32)

    def acc_body(i, carry):
        idx = didx_v[pl.ds(i * 16, 16)]
        plsc.addupdate_scatter(deg_v, [idx], ones16)
        return carry

    lax.fori_loop(0, E_PER_TILE // 16, acc_body, 0)

    pltpu.sync_copy(deg_v, stage_sh.at[sid])
    plsc.subcore_barrier()

    def zero_res(i, carry):
        res_v[pl.ds(i * 16, 16)] = zeros16
        return carry

    lax.fori_loop(0, RPT // 16, zero_res, 0)

    for r in range(NS):
        pltpu.sync_copy(stage_sh.at[r, pl.ds(sid * RPT, RPT)], row_v)

        def add_body(ci, carry):
            sl = pl.ds(ci * 16, 16)
            res_v[sl] = res_v[sl] + row_v[sl]
            return carry

        lax.fori_loop(0, RPT // 16, add_body, 0)

    pltpu.sync_copy(res_v, out_hbm.at[cid, pl.ds(sid * RPT, RPT)])


_deg_kernel = pl.kernel(
    _deg_body,
    out_type=jax.ShapeDtypeStruct((NC, NPAD), jnp.float32),
    mesh=_mesh,
    scratch_types=[
        pltpu.VMEM((E_PER_TILE,), jnp.int32),
        pltpu.VMEM((NPAD,), jnp.float32),
        pltpu.VMEM((RPT,), jnp.float32),
        pltpu.VMEM((RPT,), jnp.float32),
        pltpu.VMEM_SHARED((NS, NPAD), jnp.float32),
    ],
    compiler_params=pltpu.CompilerParams(needs_layout_passes=False),
)


def _agg_body(y_hbm, eidx_hbm, out_hbm, idx_v, *rest):
    sbufs = rest[:NBUF]
    dbufs = rest[NBUF:2 * NBUF]
    gbufs = rest[2 * NBUF:3 * NBUF]
    fbufs = rest[3 * NBUF:4 * NBUF]
    acc_sh = rest[4 * NBUF]
    sems = rest[4 * NBUF + 1:]
    gsems = sems[:NBUF]
    ssems = sems[NBUF:]
    cid = lax.axis_index("c")
    sid = lax.axis_index("s")
    rbase = sid * RPT
    bias = (cid * NPAD).astype(jnp.int32)

    pltpu.sync_copy(eidx_hbm.at[sid], idx_v)

    mask_hi = jnp.int32(-65536)
    mask_lo = jnp.int32(65535)

    def unpack_idx(j, b):
        for c in range(K // 16):
            sl = pl.ds(c * 16, 16)
            w = idx_v[j, sl]
            sbufs[b][sl] = (w & mask_lo) + bias
            dbufs[b][sl] = lax.shift_right_logical(w, 16)

    def convert(gb, fb):
        def row_body(r, carry):
            for h in range(DH // 32):
                w = gb[r, pl.ds(h * 16, 16)]
                lo = lax.bitcast_convert_type(w << 16, jnp.float32)
                hi = lax.bitcast_convert_type(w & mask_hi, jnp.float32)
                fb[r, pl.ds(h * 32, 16)] = lo
                fb[r, pl.ds(h * 32 + 16, 16)] = hi
            return carry

        lax.fori_loop(0, K, row_body, 0)

    gb0 = gbufs[0]
    fb0 = fbufs[0]
    for b in range(RPT // K):
        pltpu.sync_copy(y_hbm.at[pl.ds(cid * NPAD + rbase + b * K, K)], gb0)
        convert(gb0, fb0)
        pltpu.sync_copy(fb0, acc_sh.at[pl.ds(rbase + b * K, K)])

    plsc.subcore_barrier()

    def g_issue(j, b):
        pltpu.async_copy(y_hbm.at[sbufs[b]], gbufs[b], gsems[b])

    def g_wait(b):
        pltpu.make_async_copy(y_hbm.at[sbufs[b]], gbufs[b], gsems[b]).wait()

    def s_issue(b):
        pltpu.async_copy(fbufs[b], acc_sh.at[dbufs[b]], ssems[b], add=True)

    def s_wait(b):
        pltpu.make_async_copy(fbufs[b], acc_sh.at[dbufs[b]], ssems[b]).wait()

    def drain(b):
        g_wait(b)
        convert(gbufs[b], fbufs[b])
        s_issue(b)

    unpack_idx(0, 0)
    g_issue(0, 0)
    unpack_idx(1, 1)
    g_issue(1, 1)
    drain(0)

    def group_body(gi, carry):
        for b in range(NBUF):
            j = gi * NBUF + b
            s_wait(b)
            unpack_idx(j, b)
            g_issue(j, b)
            drain(1 - b)
        return carry

    lax.fori_loop(1, NCHUNK // NBUF, group_body, 0)

    drain(1)
    s_wait(0)
    s_wait(1)

    plsc.subcore_barrier()

    for b in range(RPT // K):
        sl = pl.ds(rbase + b * K, K)
        pltpu.sync_copy(acc_sh.at[sl], fb0)
        pltpu.sync_copy(fb0, out_hbm.at[cid, sl])


_agg_kernel = pl.kernel(
    _agg_body,
    out_type=jax.ShapeDtypeStruct((NC, NPAD, DH), jnp.float32),
    mesh=_mesh,
    scratch_types=[
        pltpu.VMEM((NCHUNK, K), jnp.int32),
    ]
    + [pltpu.VMEM((K,), jnp.int32)] * NBUF
    + [pltpu.VMEM((K,), jnp.int32)] * NBUF
    + [pltpu.VMEM((K, DH // 2), jnp.int32)] * NBUF
    + [pltpu.VMEM((K, DH), jnp.float32)] * NBUF
    + [
        pltpu.VMEM_SHARED((NPAD, DH), jnp.float32),
    ]
    + [pltpu.SemaphoreType.DMA] * (2 * NBUF),
    compiler_params=pltpu.CompilerParams(use_tc_tiling_on_sc=False),
)


BLK = 1024
GRID = NPAD // BLK


def _mm_body(x_ref, w_ref, o_ref):
    o_ref[...] = jnp.dot(x_ref[...], w_ref[...], preferred_element_type=jnp.float32)


def _tc_matmul(x, w):
    return pl.pallas_call(
        _mm_body,
        grid=(GRID,),
        in_specs=[
            pl.BlockSpec((BLK, D), lambda i: (i, 0)),
            pl.BlockSpec((D, D), lambda i: (0, 0)),
        ],
        out_specs=pl.BlockSpec((BLK, D), lambda i: (i, 0)),
        out_shape=jax.ShapeDtypeStruct((NPAD, D), jnp.float32),
    )(x, w)


def _scale_body(degT_ref, xw_ref, y_ref, dinv_ref):
    d = degT_ref[...]
    dinv = lax.rsqrt(d[:, 0:1] + d[:, 1:2] + 1.0)
    dinv_ref[...] = dinv
    y = (xw_ref[...] * dinv).astype(jnp.bfloat16)
    y_ref[0] = y[:, :DH]
    y_ref[1] = y[:, DH:]


def _tc_scale(degT, xw):
    return pl.pallas_call(
        _scale_body,
        grid=(GRID,),
        in_specs=[
            pl.BlockSpec((BLK, 2), lambda i: (i, 0)),
            pl.BlockSpec((BLK, D), lambda i: (i, 0)),
        ],
        out_specs=[
            pl.BlockSpec((NC, BLK, DH), lambda i: (0, i, 0)),
            pl.BlockSpec((BLK, 1), lambda i: (i, 0)),
        ],
        out_shape=[
            jax.ShapeDtypeStruct((NC, NPAD, DH), jnp.bfloat16),
            jax.ShapeDtypeStruct((NPAD, 1), jnp.float32),
        ],
    )(degT, xw)


def _mid_body(p0_ref, p1_ref, dinv_ref, b1p_ref, w2p_ref, y2_ref):
    dinv = dinv_ref[...]
    ph = jnp.concatenate([p0_ref[...], p1_ref[...]], axis=1)
    h = jnp.maximum(ph * dinv + b1p_ref[...], 0.0)
    y2 = (jnp.dot(h, w2p_ref[...], preferred_element_type=jnp.float32)
          * dinv).astype(jnp.bfloat16)
    y2_ref[0] = y2[:, :DH]
    y2_ref[1] = y2[:, DH:]


def _tc_mid(p0, p1, dinv, b1p, w2p):
    return pl.pallas_call(
        _mid_body,
        grid=(GRID,),
        in_specs=[
            pl.BlockSpec((BLK, DH), lambda i: (i, 0)),
            pl.BlockSpec((BLK, DH), lambda i: (i, 0)),
            pl.BlockSpec((BLK, 1), lambda i: (i, 0)),
            pl.BlockSpec((1, D), lambda i: (0, 0)),
            pl.BlockSpec((D, D), lambda i: (0, 0)),
        ],
        out_specs=pl.BlockSpec((NC, BLK, DH), lambda i: (0, i, 0)),
        out_shape=jax.ShapeDtypeStruct((NC, NPAD, DH), jnp.bfloat16),
    )(p0, p1, dinv, b1p, w2p)


def _out_body(q0_ref, q1_ref, dinv_ref, b2p_ref, o_ref):
    q = jnp.concatenate([q0_ref[...], q1_ref[...]], axis=1)
    o_ref[...] = q * dinv_ref[...] + b2p_ref[...]


def _tc_out(q0, q1, dinv, b2p):
    return pl.pallas_call(
        _out_body,
        grid=(GRID,),
        in_specs=[
            pl.BlockSpec((BLK, DH), lambda i: (i, 0)),
            pl.BlockSpec((BLK, DH), lambda i: (i, 0)),
            pl.BlockSpec((BLK, 1), lambda i: (i, 0)),
            pl.BlockSpec((1, D), lambda i: (0, 0)),
        ],
        out_specs=pl.BlockSpec((BLK, D), lambda i: (i, 0)),
        out_shape=jax.ShapeDtypeStruct((NPAD, D), jnp.float32),
    )(q0, q1, dinv, b2p)


def kernel(x, edge_index, W1, b1, W2, b2):
    src = edge_index[0].astype(jnp.int32)
    dst = edge_index[1].astype(jnp.int32)
    xp = jnp.pad(x, ((0, NPAD - N_NODES), (0, 0)))

    e_pad = NS * EPT_PAD - N_EDGES
    srcp = jnp.concatenate([src, jnp.zeros((e_pad,), jnp.int32)])
    dstp = jnp.concatenate([dst, jnp.full((e_pad,), NPAD - 1, jnp.int32)])
    eidx = (srcp | (dstp << 16)).reshape(NS, NCHUNK, K)

    permf = jnp.asarray(PERM_F)
    b1p = b1[permf].reshape(1, D)
    b2p = b2[permf].reshape(1, D)
    w2p = W2[permf, :]

    deg = _deg_kernel(dst)
    xw1 = _tc_matmul(xp, W1)
    y1, dinv = _tc_scale(deg.T, xw1)

    def _pack(y_bf):
        return lax.bitcast_convert_type(
            y_bf.reshape(NC * NPAD, DH // 2, 2), jnp.int32)

    p = _agg_kernel(_pack(y1), eidx)
    y2 = _tc_mid(p[0], p[1], dinv, b1p, w2p)

    q = _agg_kernel(_pack(y2), eidx)
    outp = _tc_out(q[0], q[1], dinv, b2p)
    return outp[:N_NODES, jnp.asarray(INV_PERM_F)]


# speedup vs baseline: 1.1609x; 1.1609x over previous
import numpy as np
import jax
import jax.numpy as jnp
from jax import lax
from jax.experimental import pallas as pl
from jax.experimental.pallas import tpu as pltpu
from jax.experimental.pallas import tpu_sc as plsc

N_NODES = 10000
D = 128
DH = D // 2
N_EDGES = 320000

NPAD = 10240
NC, NS = 2, 16
NW = NC * NS
E_PER_TILE = N_EDGES // NW
K = 64
NCHUNK = 320
EPT_PAD = NCHUNK * K
NBUF = 4
STAG = 2
RPT = NPAD // NS

_p = []
for _h in range(DH // 32):
    _p += [_h * 32 + 2 * _r for _r in range(16)]
    _p += [_h * 32 + 2 * _r + 1 for _r in range(16)]
PERM_H = np.array(_p, dtype=np.int32)
PERM_F = np.concatenate([PERM_H, DH + PERM_H])
INV_PERM_F = np.argsort(PERM_F)

_mesh = plsc.VectorSubcoreMesh(core_axis_name="c", subcore_axis_name="s")


def _deg_body(dst_hbm, out_hbm, didx_v, deg_v, row_v, res_v, stage_sh):
    cid = lax.axis_index("c")
    sid = lax.axis_index("s")
    g = cid * NS + sid

    zeros16 = jnp.zeros((16,), jnp.float32)

    def zero_deg(i, carry):
        deg_v[pl.ds(i * 16, 16)] = zeros16
        return carry

    lax.fori_loop(0, NPAD // 16, zero_deg, 0)

    pltpu.sync_copy(dst_hbm.at[pl.ds(g * E_PER_TILE, E_PER_TILE)], didx_v)

    ones16 = jnp.ones((16,), jnp.float32)

    def acc_body(i, carry):
        idx = didx_v[pl.ds(i * 16, 16)]
        plsc.addupdate_scatter(deg_v, [idx], ones16)
        return carry

    lax.fori_loop(0, E_PER_TILE // 16, acc_body, 0)

    pltpu.sync_copy(deg_v, stage_sh.at[sid])
    plsc.subcore_barrier()

    def zero_res(i, carry):
        res_v[pl.ds(i * 16, 16)] = zeros16
        return carry

    lax.fori_loop(0, RPT // 16, zero_res, 0)

    for r in range(NS):
        pltpu.sync_copy(stage_sh.at[r, pl.ds(sid * RPT, RPT)], row_v)

        def add_body(ci, carry):
            sl = pl.ds(ci * 16, 16)
            res_v[sl] = res_v[sl] + row_v[sl]
            return carry

        lax.fori_loop(0, RPT // 16, add_body, 0)

    pltpu.sync_copy(res_v, out_hbm.at[cid, pl.ds(sid * RPT, RPT)])


_deg_kernel = pl.kernel(
    _deg_body,
    out_type=jax.ShapeDtypeStruct((NC, NPAD), jnp.float32),
    mesh=_mesh,
    scratch_types=[
        pltpu.VMEM((E_PER_TILE,), jnp.int32),
        pltpu.VMEM((NPAD,), jnp.float32),
        pltpu.VMEM((RPT,), jnp.float32),
        pltpu.VMEM((RPT,), jnp.float32),
        pltpu.VMEM_SHARED((NS, NPAD), jnp.float32),
    ],
    compiler_params=pltpu.CompilerParams(needs_layout_passes=False),
)


def _agg_body(y_hbm, eidx_hbm, out_hbm, idx_v, *rest):
    sbufs = rest[:NBUF]
    dbufs = rest[NBUF:2 * NBUF]
    gbufs = rest[2 * NBUF:3 * NBUF]
    fbufs = rest[3 * NBUF:4 * NBUF]
    acc_sh = rest[4 * NBUF]
    sems = rest[4 * NBUF + 1:]
    gsems = sems[:NBUF]
    ssems = sems[NBUF:]
    cid = lax.axis_index("c")
    sid = lax.axis_index("s")
    rbase = sid * RPT
    bias = (cid * NPAD).astype(jnp.int32)

    pltpu.sync_copy(eidx_hbm.at[sid], idx_v)

    mask_hi = jnp.int32(-65536)
    mask_lo = jnp.int32(65535)

    def unpack_idx(j, b):
        for c in range(K // 16):
            sl = pl.ds(c * 16, 16)
            w = idx_v[j, sl]
            sbufs[b][sl] = (w & mask_lo) + bias
            dbufs[b][sl] = lax.shift_right_logical(w, 16)

    def convert(gb, fb):
        @plsc.parallel_loop(0, K, unroll=4)
        def row_body(r):
            for h in range(DH // 32):
                w = gb[r, pl.ds(h * 16, 16)]
                lo = lax.bitcast_convert_type(w << 16, jnp.float32)
                hi = lax.bitcast_convert_type(w & mask_hi, jnp.float32)
                fb[r, pl.ds(h * 32, 16)] = lo
                fb[r, pl.ds(h * 32 + 16, 16)] = hi

    gb0 = gbufs[0]
    fb0 = fbufs[0]
    for b in range(RPT // K):
        pltpu.sync_copy(y_hbm.at[pl.ds(cid * NPAD + rbase + b * K, K)], gb0)
        convert(gb0, fb0)
        pltpu.sync_copy(fb0, acc_sh.at[pl.ds(rbase + b * K, K)])

    plsc.subcore_barrier()

    def g_issue(b):
        pltpu.async_copy(y_hbm.at[sbufs[b]], gbufs[b], gsems[b])

    def g_wait(b):
        pltpu.make_async_copy(y_hbm.at[sbufs[b]], gbufs[b], gsems[b]).wait()

    def s_issue(b):
        pltpu.async_copy(fbufs[b], acc_sh.at[dbufs[b]], ssems[b], add=True)

    def s_wait(b):
        pltpu.make_async_copy(fbufs[b], acc_sh.at[dbufs[b]], ssems[b]).wait()

    def drain(b):
        g_wait(b)
        convert(gbufs[b], fbufs[b])
        s_issue(b)

    for p in range(NBUF):
        unpack_idx(p, p)
        g_issue(p)
        if p >= STAG:
            drain(p - STAG)

    def group_body(gi, carry):
        for b in range(NBUF):
            j = gi * NBUF + b
            s_wait(b)
            unpack_idx(j, b)
            g_issue(b)
            drain((b - STAG) % NBUF)
        return carry

    lax.fori_loop(1, NCHUNK // NBUF, group_body, 0)

    for t in range(NCHUNK - STAG, NCHUNK):
        drain(t % NBUF)
    for b in range(NBUF):
        s_wait(b)

    plsc.subcore_barrier()

    for b in range(RPT // K):
        sl = pl.ds(rbase + b * K, K)
        pltpu.sync_copy(acc_sh.at[sl], fb0)
        pltpu.sync_copy(fb0, out_hbm.at[cid, sl])


_agg_kernel = pl.kernel(
    _agg_body,
    out_type=jax.ShapeDtypeStruct((NC, NPAD, DH), jnp.float32),
    mesh=_mesh,
    scratch_types=[
        pltpu.VMEM((NCHUNK, K), jnp.int32),
    ]
    + [pltpu.VMEM((K,), jnp.int32)] * NBUF
    + [pltpu.VMEM((K,), jnp.int32)] * NBUF
    + [pltpu.VMEM((K, DH // 2), jnp.int32)] * NBUF
    + [pltpu.VMEM((K, DH), jnp.float32)] * NBUF
    + [
        pltpu.VMEM_SHARED((NPAD, DH), jnp.float32),
    ]
    + [pltpu.SemaphoreType.DMA] * (2 * NBUF),
    compiler_params=pltpu.CompilerParams(use_tc_tiling_on_sc=False),
)


BLK = 1024
GRID = NPAD // BLK


def _mm_body(x_ref, w_ref, o_ref):
    o_ref[...] = jnp.dot(x_ref[...], w_ref[...], preferred_element_type=jnp.float32)


def _tc_matmul(x, w):
    return pl.pallas_call(
        _mm_body,
        grid=(GRID,),
        in_specs=[
            pl.BlockSpec((BLK, D), lambda i: (i, 0)),
            pl.BlockSpec((D, D), lambda i: (0, 0)),
        ],
        out_specs=pl.BlockSpec((BLK, D), lambda i: (i, 0)),
        out_shape=jax.ShapeDtypeStruct((NPAD, D), jnp.float32),
    )(x, w)


def _scale_body(degT_ref, xw_ref, y_ref, dinv_ref):
    d = degT_ref[...]
    dinv = lax.rsqrt(d[:, 0:1] + d[:, 1:2] + 1.0)
    dinv_ref[...] = dinv
    y = (xw_ref[...] * dinv).astype(jnp.bfloat16)
    y_ref[0] = y[:, :DH]
    y_ref[1] = y[:, DH:]


def _tc_scale(degT, xw):
    return pl.pallas_call(
        _scale_body,
        grid=(GRID,),
        in_specs=[
            pl.BlockSpec((BLK, 2), lambda i: (i, 0)),
            pl.BlockSpec((BLK, D), lambda i: (i, 0)),
        ],
        out_specs=[
            pl.BlockSpec((NC, BLK, DH), lambda i: (0, i, 0)),
            pl.BlockSpec((BLK, 1), lambda i: (i, 0)),
        ],
        out_shape=[
            jax.ShapeDtypeStruct((NC, NPAD, DH), jnp.bfloat16),
            jax.ShapeDtypeStruct((NPAD, 1), jnp.float32),
        ],
    )(degT, xw)


def _mid_body(p0_ref, p1_ref, dinv_ref, b1p_ref, w2p_ref, y2_ref):
    dinv = dinv_ref[...]
    ph = jnp.concatenate([p0_ref[...], p1_ref[...]], axis=1)
    h = jnp.maximum(ph * dinv + b1p_ref[...], 0.0)
    y2 = (jnp.dot(h, w2p_ref[...], preferred_element_type=jnp.float32)
          * dinv).astype(jnp.bfloat16)
    y2_ref[0] = y2[:, :DH]
    y2_ref[1] = y2[:, DH:]


def _tc_mid(p0, p1, dinv, b1p, w2p):
    return pl.pallas_call(
        _mid_body,
        grid=(GRID,),
        in_specs=[
            pl.BlockSpec((BLK, DH), lambda i: (i, 0)),
            pl.BlockSpec((BLK, DH), lambda i: (i, 0)),
            pl.BlockSpec((BLK, 1), lambda i: (i, 0)),
            pl.BlockSpec((1, D), lambda i: (0, 0)),
            pl.BlockSpec((D, D), lambda i: (0, 0)),
        ],
        out_specs=pl.BlockSpec((NC, BLK, DH), lambda i: (0, i, 0)),
        out_shape=jax.ShapeDtypeStruct((NC, NPAD, DH), jnp.bfloat16),
    )(p0, p1, dinv, b1p, w2p)


def _out_body(q0_ref, q1_ref, dinv_ref, b2p_ref, o_ref):
    q = jnp.concatenate([q0_ref[...], q1_ref[...]], axis=1)
    o_ref[...] = q * dinv_ref[...] + b2p_ref[...]


def _tc_out(q0, q1, dinv, b2p):
    return pl.pallas_call(
        _out_body,
        grid=(GRID,),
        in_specs=[
            pl.BlockSpec((BLK, DH), lambda i: (i, 0)),
            pl.BlockSpec((BLK, DH), lambda i: (i, 0)),
            pl.BlockSpec((BLK, 1), lambda i: (i, 0)),
            pl.BlockSpec((1, D), lambda i: (0, 0)),
        ],
        out_specs=pl.BlockSpec((BLK, D), lambda i: (i, 0)),
        out_shape=jax.ShapeDtypeStruct((NPAD, D), jnp.float32),
    )(q0, q1, dinv, b2p)


def kernel(x, edge_index, W1, b1, W2, b2):
    src = edge_index[0].astype(jnp.int32)
    dst = edge_index[1].astype(jnp.int32)
    xp = jnp.pad(x, ((0, NPAD - N_NODES), (0, 0)))

    e_pad = NS * EPT_PAD - N_EDGES
    srcp = jnp.concatenate([src, jnp.zeros((e_pad,), jnp.int32)])
    dstp = jnp.concatenate([dst, jnp.full((e_pad,), NPAD - 1, jnp.int32)])
    eidx = (srcp | (dstp << 16)).reshape(NS, NCHUNK, K)

    permf = jnp.asarray(PERM_F)
    b1p = b1[permf].reshape(1, D)
    b2p = b2[permf].reshape(1, D)
    w2p = W2[permf, :]

    deg = _deg_kernel(dst)
    xw1 = _tc_matmul(xp, W1)
    y1, dinv = _tc_scale(deg.T, xw1)

    def _pack(y_bf):
        return lax.bitcast_convert_type(
            y_bf.reshape(NC * NPAD, DH // 2, 2), jnp.int32)

    p = _agg_kernel(_pack(y1), eidx)
    y2 = _tc_mid(p[0], p[1], dinv, b1p, w2p)

    q = _agg_kernel(_pack(y2), eidx)
    outp = _tc_out(q[0], q[1], dinv, b2p)
    return outp[:N_NODES, jnp.asarray(INV_PERM_F)]


# speedup vs baseline: 2.3189x; 1.9975x over previous
import numpy as np
import jax
import jax.numpy as jnp
from jax import lax
from jax.experimental import pallas as pl
from jax.experimental.pallas import tpu as pltpu
from jax.experimental.pallas import tpu_sc as plsc

N_NODES = 10000
D = 128
DH = D // 2
N_EDGES = 320000

NPAD = 10240
NC, NS = 2, 16
NW = NC * NS
E_PER_TILE = N_EDGES // NW
K = 64
NCHUNK = 320
EPT_PAD = NCHUNK * K
NBUF = 4
STAG = 2
RPT = NPAD // NS

_p = []
for _h in range(DH // 32):
    _p += [_h * 32 + 2 * _r for _r in range(16)]
    _p += [_h * 32 + 2 * _r + 1 for _r in range(16)]
PERM_H = np.array(_p, dtype=np.int32)
PERM_F = np.concatenate([PERM_H, DH + PERM_H])
INV_PERM_F = np.argsort(PERM_F)

_mesh = plsc.VectorSubcoreMesh(core_axis_name="c", subcore_axis_name="s")


def _deg_body(dst_hbm, out_hbm, didx_v, deg_v, row_v, res_v, stage_sh):
    cid = lax.axis_index("c")
    sid = lax.axis_index("s")
    g = cid * NS + sid

    zeros16 = jnp.zeros((16,), jnp.float32)

    def zero_deg(i, carry):
        deg_v[pl.ds(i * 16, 16)] = zeros16
        return carry

    lax.fori_loop(0, NPAD // 16, zero_deg, 0)

    pltpu.sync_copy(dst_hbm.at[pl.ds(g * E_PER_TILE, E_PER_TILE)], didx_v)

    ones16 = jnp.ones((16,), jnp.float32)

    def acc_body(i, carry):
        idx = didx_v[pl.ds(i * 16, 16)]
        plsc.addupdate_scatter(deg_v, [idx], ones16)
        return carry

    lax.fori_loop(0, E_PER_TILE // 16, acc_body, 0)

    pltpu.sync_copy(deg_v, stage_sh.at[sid])
    plsc.subcore_barrier()

    def zero_res(i, carry):
        res_v[pl.ds(i * 16, 16)] = zeros16
        return carry

    lax.fori_loop(0, RPT // 16, zero_res, 0)

    for r in range(NS):
        pltpu.sync_copy(stage_sh.at[r, pl.ds(sid * RPT, RPT)], row_v)

        def add_body(ci, carry):
            sl = pl.ds(ci * 16, 16)
            res_v[sl] = res_v[sl] + row_v[sl]
            return carry

        lax.fori_loop(0, RPT // 16, add_body, 0)

    pltpu.sync_copy(res_v, out_hbm.at[cid, pl.ds(sid * RPT, RPT)])


_deg_kernel = pl.kernel(
    _deg_body,
    out_type=jax.ShapeDtypeStruct((NC, NPAD), jnp.float32),
    mesh=_mesh,
    scratch_types=[
        pltpu.VMEM((E_PER_TILE,), jnp.int32),
        pltpu.VMEM((NPAD,), jnp.float32),
        pltpu.VMEM((RPT,), jnp.float32),
        pltpu.VMEM((RPT,), jnp.float32),
        pltpu.VMEM_SHARED((NS, NPAD), jnp.float32),
    ],
    compiler_params=pltpu.CompilerParams(needs_layout_passes=False),
)


def _agg_body(y_hbm, eidx_hbm, out_hbm, idx_v, *rest):
    sbufs = rest[:NBUF]
    dbufs = rest[NBUF:2 * NBUF]
    gbufs = rest[2 * NBUF:3 * NBUF]
    fbufs = rest[3 * NBUF:4 * NBUF]
    acc_sh = rest[4 * NBUF]
    sems = rest[4 * NBUF + 1:]
    gsems = sems[:NBUF]
    ssems = sems[NBUF:]
    cid = lax.axis_index("c")
    sid = lax.axis_index("s")
    rbase = sid * RPT
    bias = (cid * NPAD).astype(jnp.int32)

    pltpu.sync_copy(eidx_hbm.at[sid], idx_v)

    mask_hi = jnp.int32(-65536)
    mask_lo = jnp.int32(65535)

    def unpack_idx(j, b):
        for c in range(K // 16):
            sl = pl.ds(c * 16, 16)
            w = idx_v[j, sl]
            sbufs[b][sl] = (w & mask_lo) + bias
            dbufs[b][sl] = lax.shift_right_logical(w, 16)

    def convert(gb, fb):
        @plsc.parallel_loop(0, K, unroll=4)
        def row_body(r):
            for h in range(DH // 32):
                w = gb[r, pl.ds(h * 16, 16)]
                lo = lax.bitcast_convert_type(w << 16, jnp.float32)
                hi = lax.bitcast_convert_type(w & mask_hi, jnp.float32)
                fb[r, pl.ds(h * 32, 16)] = lo
                fb[r, pl.ds(h * 32 + 16, 16)] = hi

    gb0 = gbufs[0]
    fb0 = fbufs[0]
    for b in range(RPT // K):
        pltpu.sync_copy(y_hbm.at[pl.ds(cid * NPAD + rbase + b * K, K)], gb0)
        convert(gb0, fb0)
        pltpu.sync_copy(fb0, acc_sh.at[pl.ds(rbase + b * K, K)])

    plsc.subcore_barrier()

    def g_issue(b):
        pltpu.async_copy(y_hbm.at[sbufs[b]], gbufs[b], gsems[b])

    def g_wait(b):
        pltpu.make_async_copy(y_hbm.at[sbufs[b]], gbufs[b], gsems[b]).wait()

    def s_issue(b):
        pltpu.async_copy(fbufs[b], acc_sh.at[dbufs[b]], ssems[b], add=True)

    def s_wait(b):
        pltpu.make_async_copy(fbufs[b], acc_sh.at[dbufs[b]], ssems[b]).wait()

    def drain(b):
        g_wait(b)
        convert(gbufs[b], fbufs[b])
        s_issue(b)

    for p in range(NBUF):
        unpack_idx(p, p)
        g_issue(p)
        if p >= STAG:
            drain(p - STAG)

    def group_body(gi, carry):
        for b in range(NBUF):
            j = gi * NBUF + b
            s_wait(b)
            unpack_idx(j, b)
            g_issue(b)
            drain((b - STAG) % NBUF)
        return carry

    lax.fori_loop(1, NCHUNK // NBUF, group_body, 0)

    for t in range(NCHUNK - STAG, NCHUNK):
        drain(t % NBUF)
    for b in range(NBUF):
        s_wait(b)

    plsc.subcore_barrier()

    for b in range(RPT // K):
        sl = pl.ds(rbase + b * K, K)
        pltpu.sync_copy(acc_sh.at[sl], fb0)
        pltpu.sync_copy(fb0, out_hbm.at[cid, sl])


_agg_kernel = pl.kernel(
    _agg_body,
    out_type=jax.ShapeDtypeStruct((NC, NPAD, DH), jnp.float32),
    mesh=_mesh,
    scratch_types=[
        pltpu.VMEM((NCHUNK, K), jnp.int32),
    ]
    + [pltpu.VMEM((K,), jnp.int32)] * NBUF
    + [pltpu.VMEM((K,), jnp.int32)] * NBUF
    + [pltpu.VMEM((K, DH // 2), jnp.int32)] * NBUF
    + [pltpu.VMEM((K, DH), jnp.float32)] * NBUF
    + [
        pltpu.VMEM_SHARED((NPAD, DH), jnp.float32),
    ]
    + [pltpu.SemaphoreType.DMA] * (2 * NBUF),
    compiler_params=pltpu.CompilerParams(use_tc_tiling_on_sc=False),
)


BLK = 1024
GRID = NPAD // BLK


def _mm_body(x_ref, w_ref, o_ref):
    o_ref[...] = jnp.dot(x_ref[...], w_ref[...], preferred_element_type=jnp.float32)


def _tc_matmul(x, w):
    return pl.pallas_call(
        _mm_body,
        grid=(GRID,),
        in_specs=[
            pl.BlockSpec((BLK, D), lambda i: (i, 0)),
            pl.BlockSpec((D, D), lambda i: (0, 0)),
        ],
        out_specs=pl.BlockSpec((BLK, D), lambda i: (i, 0)),
        out_shape=jax.ShapeDtypeStruct((NPAD, D), jnp.float32),
    )(x, w)


def _scale_body(degT_ref, xw_ref, y_ref, dinv_ref):
    d = degT_ref[...]
    dinv = lax.rsqrt(d[:, 0:1] + d[:, 1:2] + 1.0)
    dinv_ref[...] = dinv
    y = (xw_ref[...] * dinv).astype(jnp.bfloat16)
    y_ref[0] = y[:, :DH]
    y_ref[1] = y[:, DH:]


def _tc_scale(degT, xw):
    return pl.pallas_call(
        _scale_body,
        grid=(GRID,),
        in_specs=[
            pl.BlockSpec((BLK, 2), lambda i: (i, 0)),
            pl.BlockSpec((BLK, D), lambda i: (i, 0)),
        ],
        out_specs=[
            pl.BlockSpec((NC, BLK, DH), lambda i: (0, i, 0)),
            pl.BlockSpec((BLK, 1), lambda i: (i, 0)),
        ],
        out_shape=[
            jax.ShapeDtypeStruct((NC, NPAD, DH), jnp.bfloat16),
            jax.ShapeDtypeStruct((NPAD, 1), jnp.float32),
        ],
    )(degT, xw)


def _mid_body(p0_ref, p1_ref, dinv_ref, b1_ref, w2ip_ref, y2_ref):
    dinv = dinv_ref[...]
    ph = jnp.concatenate([p0_ref[...], p1_ref[...]], axis=1)
    h = jnp.maximum(ph * dinv + b1_ref[...], 0.0)
    y2 = (jnp.dot(h, w2ip_ref[...], preferred_element_type=jnp.float32)
          * dinv).astype(jnp.bfloat16)
    y2_ref[0] = y2[:, :DH]
    y2_ref[1] = y2[:, DH:]


def _tc_mid(p0, p1, dinv, b1p, w2p):
    return pl.pallas_call(
        _mid_body,
        grid=(GRID,),
        in_specs=[
            pl.BlockSpec((BLK, DH), lambda i: (i, 0)),
            pl.BlockSpec((BLK, DH), lambda i: (i, 0)),
            pl.BlockSpec((BLK, 1), lambda i: (i, 0)),
            pl.BlockSpec((1, D), lambda i: (0, 0)),
            pl.BlockSpec((D, D), lambda i: (0, 0)),
        ],
        out_specs=pl.BlockSpec((NC, BLK, DH), lambda i: (0, i, 0)),
        out_shape=jax.ShapeDtypeStruct((NC, NPAD, DH), jnp.bfloat16),
    )(p0, p1, dinv, b1p, w2p)


def _out_body(q0_ref, q1_ref, dinv_ref, b2_ref, o_ref):
    q = jnp.concatenate([q0_ref[...], q1_ref[...]], axis=1)
    o_ref[...] = q * dinv_ref[...] + b2_ref[...]


def _tc_out(q0, q1, dinv, b2p):
    return pl.pallas_call(
        _out_body,
        grid=(GRID,),
        in_specs=[
            pl.BlockSpec((BLK, DH), lambda i: (i, 0)),
            pl.BlockSpec((BLK, DH), lambda i: (i, 0)),
            pl.BlockSpec((BLK, 1), lambda i: (i, 0)),
            pl.BlockSpec((1, D), lambda i: (0, 0)),
        ],
        out_specs=pl.BlockSpec((BLK, D), lambda i: (i, 0)),
        out_shape=jax.ShapeDtypeStruct((NPAD, D), jnp.float32),
    )(q0, q1, dinv, b2p)


def kernel(x, edge_index, W1, b1, W2, b2):
    src = edge_index[0].astype(jnp.int32)
    dst = edge_index[1].astype(jnp.int32)
    xp = jnp.pad(x, ((0, NPAD - N_NODES), (0, 0)))

    e_pad = NS * EPT_PAD - N_EDGES
    srcp = jnp.concatenate([src, jnp.zeros((e_pad,), jnp.int32)])
    dstp = jnp.concatenate([dst, jnp.full((e_pad,), NPAD - 1, jnp.int32)])
    eidx = (srcp | (dstp << 16)).reshape(NS, NCHUNK, K)

    invp = jnp.asarray(INV_PERM_F)
    w1ip = W1[:, invp]
    w2ip = W2[:, invp]

    deg = _deg_kernel(dst)
    xw1 = _tc_matmul(xp, w1ip)
    y1, dinv = _tc_scale(deg.T, xw1)

    def _pack(y_bf):
        return lax.bitcast_convert_type(
            y_bf.reshape(NC * NPAD, DH // 2, 2), jnp.int32)

    p = _agg_kernel(_pack(y1), eidx)
    y2 = _tc_mid(p[0], p[1], dinv, b1.reshape(1, D), w2ip)

    q = _agg_kernel(_pack(y2), eidx)
    outp = _tc_out(q[0], q[1], dinv, b2.reshape(1, D))
    return outp[:N_NODES]


# speedup vs baseline: 2.7148x; 1.1707x over previous
import numpy as np
import jax
import jax.numpy as jnp
from jax import lax
from jax.experimental import pallas as pl
from jax.experimental.pallas import tpu as pltpu
from jax.experimental.pallas import tpu_sc as plsc

N_NODES = 10000
D = 128
DH = D // 2
N_EDGES = 320000

NPAD = 10240
NC, NS = 2, 16
NW = NC * NS
E_PER_TILE = N_EDGES // NW
K = 64
NCHUNK = 320
EPT_PAD = NCHUNK * K
NBUF = 4
STAG = 2
RPT = NPAD // NS

_p = []
for _c in range(NC):
    for _g in range(2):
        _p += [32 * _c + 16 * _g + _l for _l in range(16)]
        _p += [64 + 32 * _c + 16 * _g + _l for _l in range(16)]
PERM_F = np.array(_p, dtype=np.int32)
INV_PERM_F = np.argsort(PERM_F)

_mesh = plsc.VectorSubcoreMesh(core_axis_name="c", subcore_axis_name="s")


def _deg_body(dst_hbm, out_hbm, didx_v, deg_v, row_v, res_v, stage_sh):
    cid = lax.axis_index("c")
    sid = lax.axis_index("s")
    g = cid * NS + sid

    zeros16 = jnp.zeros((16,), jnp.float32)

    def zero_deg(i, carry):
        deg_v[pl.ds(i * 16, 16)] = zeros16
        return carry

    lax.fori_loop(0, NPAD // 16, zero_deg, 0)

    pltpu.sync_copy(dst_hbm.at[pl.ds(g * E_PER_TILE, E_PER_TILE)], didx_v)

    ones16 = jnp.ones((16,), jnp.float32)

    def acc_body(i, carry):
        idx = didx_v[pl.ds(i * 16, 16)]
        plsc.addupdate_scatter(deg_v, [idx], ones16)
        return carry

    lax.fori_loop(0, E_PER_TILE // 16, acc_body, 0)

    pltpu.sync_copy(deg_v, stage_sh.at[sid])
    plsc.subcore_barrier()

    def zero_res(i, carry):
        res_v[pl.ds(i * 16, 16)] = zeros16
        return carry

    lax.fori_loop(0, RPT // 16, zero_res, 0)

    for r in range(NS):
        pltpu.sync_copy(stage_sh.at[r, pl.ds(sid * RPT, RPT)], row_v)

        def add_body(ci, carry):
            sl = pl.ds(ci * 16, 16)
            res_v[sl] = res_v[sl] + row_v[sl]
            return carry

        lax.fori_loop(0, RPT // 16, add_body, 0)

    pltpu.sync_copy(res_v, out_hbm.at[cid, pl.ds(sid * RPT, RPT)])


_deg_kernel = pl.kernel(
    _deg_body,
    out_type=jax.ShapeDtypeStruct((NC, NPAD), jnp.float32),
    mesh=_mesh,
    scratch_types=[
        pltpu.VMEM((E_PER_TILE,), jnp.int32),
        pltpu.VMEM((NPAD,), jnp.float32),
        pltpu.VMEM((RPT,), jnp.float32),
        pltpu.VMEM((RPT,), jnp.float32),
        pltpu.VMEM_SHARED((NS, NPAD), jnp.float32),
    ],
    compiler_params=pltpu.CompilerParams(needs_layout_passes=False),
)


def _agg_body(y_hbm, eidx_hbm, out_hbm, idx_v, *rest):
    sbufs = rest[:NBUF]
    dbufs = rest[NBUF:2 * NBUF]
    gbufs = rest[2 * NBUF:3 * NBUF]
    fbufs = rest[3 * NBUF:4 * NBUF]
    acc_sh = rest[4 * NBUF]
    sems = rest[4 * NBUF + 1:]
    gsems = sems[:NBUF]
    ssems = sems[NBUF:]
    cid = lax.axis_index("c")
    sid = lax.axis_index("s")
    rbase = sid * RPT
    bias = (cid * NPAD).astype(jnp.int32)

    pltpu.sync_copy(eidx_hbm.at[sid], idx_v)

    mask_hi = jnp.int32(-65536)
    mask_lo = jnp.int32(65535)

    def unpack_idx(j, b):
        for c in range(K // 16):
            sl = pl.ds(c * 16, 16)
            w = idx_v[j, sl]
            sbufs[b][sl] = (w & mask_lo) + bias
            dbufs[b][sl] = lax.shift_right_logical(w, 16)

    def convert(gb, fb):
        @plsc.parallel_loop(0, K, unroll=4)
        def row_body(r):
            for h in range(DH // 32):
                w = gb[r, pl.ds(h * 16, 16)]
                lo = lax.bitcast_convert_type(w << 16, jnp.float32)
                hi = lax.bitcast_convert_type(w & mask_hi, jnp.float32)
                fb[r, pl.ds(h * 32, 16)] = lo
                fb[r, pl.ds(h * 32 + 16, 16)] = hi

    gb0 = gbufs[0]
    fb0 = fbufs[0]
    for b in range(RPT // K):
        pltpu.sync_copy(y_hbm.at[pl.ds(cid * NPAD + rbase + b * K, K)], gb0)
        convert(gb0, fb0)
        pltpu.sync_copy(fb0, acc_sh.at[pl.ds(rbase + b * K, K)])

    plsc.subcore_barrier()

    def g_issue(b):
        pltpu.async_copy(y_hbm.at[sbufs[b]], gbufs[b], gsems[b])

    def g_wait(b):
        pltpu.make_async_copy(y_hbm.at[sbufs[b]], gbufs[b], gsems[b]).wait()

    def s_issue(b):
        pltpu.async_copy(fbufs[b], acc_sh.at[dbufs[b]], ssems[b], add=True)

    def s_wait(b):
        pltpu.make_async_copy(fbufs[b], acc_sh.at[dbufs[b]], ssems[b]).wait()

    def drain(b):
        g_wait(b)
        convert(gbufs[b], fbufs[b])
        s_issue(b)

    for p in range(NBUF):
        unpack_idx(p, p)
        g_issue(p)
        if p >= STAG:
            drain(p - STAG)

    def group_body(gi, carry):
        for b in range(NBUF):
            j = gi * NBUF + b
            s_wait(b)
            unpack_idx(j, b)
            g_issue(b)
            drain((b - STAG) % NBUF)
        return carry

    lax.fori_loop(1, NCHUNK // NBUF, group_body, 0)

    for t in range(NCHUNK - STAG, NCHUNK):
        drain(t % NBUF)
    for b in range(NBUF):
        s_wait(b)

    plsc.subcore_barrier()

    for b in range(RPT // K):
        sl = pl.ds(rbase + b * K, K)
        pltpu.sync_copy(acc_sh.at[sl], fb0)
        pltpu.sync_copy(fb0, out_hbm.at[cid, sl])


_agg_kernel = pl.kernel(
    _agg_body,
    out_type=jax.ShapeDtypeStruct((NC, NPAD, DH), jnp.float32),
    mesh=_mesh,
    scratch_types=[
        pltpu.VMEM((NCHUNK, K), jnp.int32),
    ]
    + [pltpu.VMEM((K,), jnp.int32)] * NBUF
    + [pltpu.VMEM((K,), jnp.int32)] * NBUF
    + [pltpu.VMEM((K, DH // 2), jnp.int32)] * NBUF
    + [pltpu.VMEM((K, DH), jnp.float32)] * NBUF
    + [
        pltpu.VMEM_SHARED((NPAD, DH), jnp.float32),
    ]
    + [pltpu.SemaphoreType.DMA] * (2 * NBUF),
    compiler_params=pltpu.CompilerParams(use_tc_tiling_on_sc=False),
)


BLK = 1024
GRID = NPAD // BLK


def _mm_body(x_ref, w_ref, o_ref):
    o_ref[...] = jnp.dot(x_ref[...], w_ref[...], preferred_element_type=jnp.float32)


def _tc_matmul(x, w):
    return pl.pallas_call(
        _mm_body,
        grid=(GRID,),
        in_specs=[
            pl.BlockSpec((BLK, D), lambda i: (i, 0)),
            pl.BlockSpec((D, D), lambda i: (0, 0)),
        ],
        out_specs=pl.BlockSpec((BLK, D), lambda i: (i, 0)),
        out_shape=jax.ShapeDtypeStruct((NPAD, D), jnp.float32),
    )(x, w)


def _pack_y(y):
    u = lax.bitcast_convert_type(y.astype(jnp.bfloat16), jnp.uint16)
    w = u[:, :DH].astype(jnp.int32) | (u[:, DH:].astype(jnp.int32) << 16)
    return w


def _scale_body(degT_ref, xw_ref, y_ref, dinv_ref):
    d = degT_ref[...]
    dinv = lax.rsqrt(d[:, 0:1] + d[:, 1:2] + 1.0)
    dinv_ref[...] = dinv
    w = _pack_y(xw_ref[...] * dinv)
    y_ref[0] = w[:, :DH // 2]
    y_ref[1] = w[:, DH // 2:]


def _tc_scale(degT, xw):
    return pl.pallas_call(
        _scale_body,
        grid=(GRID,),
        in_specs=[
            pl.BlockSpec((BLK, 2), lambda i: (i, 0)),
            pl.BlockSpec((BLK, D), lambda i: (i, 0)),
        ],
        out_specs=[
            pl.BlockSpec((NC, BLK, DH // 2), lambda i: (0, i, 0)),
            pl.BlockSpec((BLK, 1), lambda i: (i, 0)),
        ],
        out_shape=[
            jax.ShapeDtypeStruct((NC, NPAD, DH // 2), jnp.int32),
            jax.ShapeDtypeStruct((NPAD, 1), jnp.float32),
        ],
    )(degT, xw)


def _mid_body(p0_ref, p1_ref, dinv_ref, b1_ref, w2ip_ref, y2_ref):
    dinv = dinv_ref[...]
    ph = jnp.concatenate([p0_ref[...], p1_ref[...]], axis=1)
    h = jnp.maximum(ph * dinv + b1_ref[...], 0.0)
    w = _pack_y(
        jnp.dot(h, w2ip_ref[...], preferred_element_type=jnp.float32) * dinv)
    y2_ref[0] = w[:, :DH // 2]
    y2_ref[1] = w[:, DH // 2:]


def _tc_mid(p0, p1, dinv, b1p, w2p):
    return pl.pallas_call(
        _mid_body,
        grid=(GRID,),
        in_specs=[
            pl.BlockSpec((BLK, DH), lambda i: (i, 0)),
            pl.BlockSpec((BLK, DH), lambda i: (i, 0)),
            pl.BlockSpec((BLK, 1), lambda i: (i, 0)),
            pl.BlockSpec((1, D), lambda i: (0, 0)),
            pl.BlockSpec((D, D), lambda i: (0, 0)),
        ],
        out_specs=pl.BlockSpec((NC, BLK, DH // 2), lambda i: (0, i, 0)),
        out_shape=jax.ShapeDtypeStruct((NC, NPAD, DH // 2), jnp.int32),
    )(p0, p1, dinv, b1p, w2p)


def _out_body(q0_ref, q1_ref, dinv_ref, b2_ref, o_ref):
    q = jnp.concatenate([q0_ref[...], q1_ref[...]], axis=1)
    o_ref[...] = q * dinv_ref[...] + b2_ref[...]


def _tc_out(q0, q1, dinv, b2p):
    return pl.pallas_call(
        _out_body,
        grid=(GRID,),
        in_specs=[
            pl.BlockSpec((BLK, DH), lambda i: (i, 0)),
            pl.BlockSpec((BLK, DH), lambda i: (i, 0)),
            pl.BlockSpec((BLK, 1), lambda i: (i, 0)),
            pl.BlockSpec((1, D), lambda i: (0, 0)),
        ],
        out_specs=pl.BlockSpec((BLK, D), lambda i: (i, 0)),
        out_shape=jax.ShapeDtypeStruct((NPAD, D), jnp.float32),
    )(q0, q1, dinv, b2p)


def kernel(x, edge_index, W1, b1, W2, b2):
    src = edge_index[0].astype(jnp.int32)
    dst = edge_index[1].astype(jnp.int32)
    xp = jnp.pad(x, ((0, NPAD - N_NODES), (0, 0)))

    e_pad = NS * EPT_PAD - N_EDGES
    srcp = jnp.concatenate([src, jnp.zeros((e_pad,), jnp.int32)])
    dstp = jnp.concatenate([dst, jnp.full((e_pad,), NPAD - 1, jnp.int32)])
    eidx = (srcp | (dstp << 16)).reshape(NS, NCHUNK, K)

    invp = jnp.asarray(INV_PERM_F)
    w1ip = W1[:, invp]
    w2ip = W2[:, invp]

    deg = _deg_kernel(dst)
    xw1 = _tc_matmul(xp, w1ip)
    y1, dinv = _tc_scale(deg.T, xw1)

    p = _agg_kernel(y1.reshape(NC * NPAD, DH // 2), eidx)
    y2 = _tc_mid(p[0], p[1], dinv, b1.reshape(1, D), w2ip)

    q = _agg_kernel(y2.reshape(NC * NPAD, DH // 2), eidx)
    outp = _tc_out(q[0], q[1], dinv, b2.reshape(1, D))
    return outp[:N_NODES]
